# baseline (speedup 1.0000x reference)
import jax
import jax.numpy as jnp
from jax import lax
from jax.experimental import pallas as pl

R = 512


def _tc_body(idx_ref, table_ref, out_ref):
    idx = idx_ref[0]
    onehot = jnp.where(
        idx == lax.broadcasted_iota(jnp.int32, (R, 512), 1),
        jnp.float32(1), jnp.float32(0))
    out_ref[...] = jnp.dot(onehot, table_ref[...],
                           preferred_element_type=jnp.float32)


def kernel(top_vecs, position_ids, pos_table):
    del top_vecs
    b, s = position_ids.shape
    total = b * s
    n_blocks = total // R
    idx3 = position_ids.reshape(n_blocks, R, 1).astype(jnp.int32)
    out = pl.pallas_call(
        _tc_body,
        grid=(n_blocks,),
        in_specs=[
            pl.BlockSpec((1, R, 1), lambda i: (i, 0, 0)),
            pl.BlockSpec((512, 128), lambda i: (0, 0)),
        ],
        out_specs=pl.BlockSpec((R, 128), lambda i: (i, 0)),
        out_shape=jax.ShapeDtypeStruct((total, 128), jnp.float32),
    )(idx3, pos_table)
    return out.reshape(b, s, 128)
